# baseline (speedup 1.0000x reference)
import functools

import jax
import jax.numpy as jnp
from jax import lax
from jax.experimental import pallas as pl
from jax.experimental.pallas import tpu as pltpu
from jax.experimental.pallas import tpu_sc as plsc

N = 10000
D = 256
HW = 128
E = 64000

NC = 2
NS = 16
HALF = N // NC
ACC_ROWS = 5120
GARBAGE = HALF
ZROWS = ACC_ROWS // NS
EPT = E // NS
K = 80
NCHUNK = EPT // K
OUT_T = 5
OUT_ROWS = HALF // OUT_T
NBUF = 2


def _sc_segsum(compute_deg):
    mesh = plsc.VectorSubcoreMesh(core_axis_name="c", subcore_axis_name="s")
    out_type = [jax.ShapeDtypeStruct((3, N, D), jnp.float32)]
    if compute_deg:
        out_type += [jax.ShapeDtypeStruct((3, N, HW), jnp.float32)]
    scratch = [
        pltpu.VMEM((NCHUNK, K), jnp.int32),
        pltpu.VMEM((NCHUNK, K), jnp.int32),
        pltpu.VMEM((NCHUNK, K), jnp.int32),
    ] + [pltpu.VMEM((K, HW), jnp.float32) for _ in range(NBUF)] + [
        pltpu.VMEM((K, HW), jnp.float32),
        pltpu.VMEM_SHARED((ACC_ROWS, HW), jnp.float32),
    ] + [pltpu.SemaphoreType.DMA for _ in range(2 * NBUF)]

    @functools.partial(pl.kernel, mesh=mesh, out_type=out_type,
                       scratch_types=scratch)
    def k(feat_h, src_h, dst_h, z_h, o_h, *rest):
        agg_h = rest[0]
        deg_h = rest[1] if compute_deg else None
        sc = rest[2 if compute_deg else 1:]
        srcv, dstv, ldv = sc[0:3]
        rbufs = sc[3:3 + NBUF]
        onesb, acc = sc[3 + NBUF:5 + NBUF]
        gsems = sc[5 + NBUF:5 + 2 * NBUF]
        ssems = sc[5 + 2 * NBUF:5 + 3 * NBUF]

        c = lax.axis_index("c")
        s = lax.axis_index("s")
        lo = c * HALF
        bufs = tuple(zip(rbufs, gsems, ssems))
        dummy = feat_h.at[pl.ds(0, K), pl.ds(0, HW)]

        def zero_acc():
            pltpu.sync_copy(z_h, acc.at[pl.ds(s * ZROWS, ZROWS)])

        WIN = 8

        def deg_pass(r, nt):
            zero_acc()
            plsc.subcore_barrier()

            def dbody(i, carry):
                pltpu.async_copy(onesb, acc.at[ldv.at[i]], ssems[0], add=True)

                @pl.when(i >= WIN)
                def _():
                    pltpu.make_async_copy(dummy, onesb, ssems[0]).wait()

                return carry

            lax.fori_loop(0, nt, dbody, 0)

            def ddrain(i, carry):
                pltpu.make_async_copy(dummy, onesb, ssems[0]).wait()
                return carry

            lax.fori_loop(0, WIN, ddrain, 0)
            plsc.subcore_barrier()

            @pl.when(s < OUT_T)
            def _():
                r0 = pl.multiple_of(s * OUT_ROWS, 8)
                pltpu.sync_copy(acc.at[pl.ds(r0, OUT_ROWS)],
                                deg_h.at[r, pl.ds(lo + r0, OUT_ROWS)])

            plsc.subcore_barrier()

        if compute_deg:
            pltpu.sync_copy(o_h, onesb)

        for r in range(3):
            pltpu.sync_copy(src_h.at[r, s], srcv)
            pltpu.sync_copy(dst_h.at[r, s], dstv)

            def routemk(i, carry2):
                for g in range(K // 16):
                    dv = dstv[i, pl.ds(g * 16, 16)]
                    m = (dv >= lo) & (dv < lo + HALF)
                    ldv[i, pl.ds(g * 16, 16)] = jnp.where(m, dv - lo, GARBAGE)
                return carry2

            lax.fori_loop(0, NCHUNK, routemk, 0)
            nt = NCHUNK

            if compute_deg:
                deg_pass(r, nt)

            for cb in (0, HW):
                def issue_gather(i, buf):
                    r_, gsem, _ = buf
                    pltpu.async_copy(feat_h.at[srcv.at[i], pl.ds(cb, HW)],
                                     r_, gsem)

                def wait_gather(buf):
                    r_, gsem, _ = buf
                    pltpu.make_async_copy(dummy, r_, gsem).wait()

                def issue_scatter(i, buf):
                    r_, _, ssem = buf
                    pltpu.async_copy(r_, acc.at[ldv.at[i]], ssem, add=True)

                def wait_scatter(buf):
                    r_, _, ssem = buf
                    pltpu.make_async_copy(dummy, r_, ssem).wait()

                zero_acc()
                plsc.subcore_barrier()

                for p in range(NBUF):
                    issue_gather(p, bufs[p])

                def body(j, carry):
                    for p in range(NBUF):
                        i = NBUF * j + p
                        wait_gather(bufs[p])
                        issue_scatter(i, bufs[p])

                        @pl.when(i + NBUF < nt)
                        def _():
                            wait_scatter(bufs[p])
                            issue_gather(i + NBUF, bufs[p])

                    return carry

                lax.fori_loop(0, lax.div(nt, NBUF), body, 0)
                for p in range(NBUF):
                    wait_scatter(bufs[p])
                plsc.subcore_barrier()

                @pl.when(s < OUT_T)
                def _():
                    r0 = pl.multiple_of(s * OUT_ROWS, 8)
                    pltpu.sync_copy(acc.at[pl.ds(r0, OUT_ROWS)],
                                    agg_h.at[r, pl.ds(lo + r0, OUT_ROWS),
                                             pl.ds(cb, HW)])

                plsc.subcore_barrier()

    return k


_sc_l0 = _sc_segsum(compute_deg=True)
_sc_l1 = _sc_segsum(compute_deg=False)


BLK = 512
_GRID = (N + BLK - 1) // BLK


def _tc_body(a0, a1, a2, g0, g1, g2, w0, w1, w2, b0, b1, b2, o):
    inv0 = 1.0 / jnp.maximum(g0[...][:, 0:1], 1.0)
    inv1 = 1.0 / jnp.maximum(g1[...][:, 0:1], 1.0)
    inv2 = 1.0 / jnp.maximum(g2[...][:, 0:1], 1.0)
    acc = jnp.dot(a0[...] * inv0, w0[...], preferred_element_type=jnp.float32)
    acc = acc + jnp.dot(a1[...] * inv1, w1[...], preferred_element_type=jnp.float32)
    acc = acc + jnp.dot(a2[...] * inv2, w2[...], preferred_element_type=jnp.float32)
    acc = acc + (b0[...] + b1[...] + b2[...])
    o[...] = jnp.maximum(acc, 0.0)


def _tc_layer(aggs, degs, ws, bs):
    row = lambda i: (i, 0)
    fix = lambda i: (0, 0)
    return pl.pallas_call(
        _tc_body,
        grid=(_GRID,),
        in_specs=[
            pl.BlockSpec((BLK, D), row), pl.BlockSpec((BLK, D), row),
            pl.BlockSpec((BLK, D), row),
            pl.BlockSpec((BLK, HW), row), pl.BlockSpec((BLK, HW), row),
            pl.BlockSpec((BLK, HW), row),
            pl.BlockSpec((D, D), fix), pl.BlockSpec((D, D), fix),
            pl.BlockSpec((D, D), fix),
            pl.BlockSpec((1, D), fix), pl.BlockSpec((1, D), fix),
            pl.BlockSpec((1, D), fix),
        ],
        out_specs=pl.BlockSpec((BLK, D), row),
        out_shape=jax.ShapeDtypeStruct((N, D), jnp.float32),
    )(*aggs, *degs, *ws, bs[0].reshape(1, D), bs[1].reshape(1, D),
      bs[2].reshape(1, D))


def kernel(x, edge_index_r0, edge_index_r1, edge_index_r2,
           W0_r0, b0_r0, W0_r1, b0_r1, W0_r2, b0_r2,
           W1_r0, b1_r0, W1_r1, b1_r1, W1_r2, b1_r2):
    ei = jnp.stack([edge_index_r0, edge_index_r1, edge_index_r2])
    src = ei[:, 0].reshape(3, NS, NCHUNK, K)
    dst = ei[:, 1].reshape(3, NS, NCHUNK, K)
    z = jnp.zeros((ZROWS, HW), jnp.float32)
    o = jnp.ones((K, HW), jnp.float32)

    agg, deg = _sc_l0(x, src, dst, z, o)
    aggs, degs = (agg[0], agg[1], agg[2]), (deg[0], deg[1], deg[2])
    h = _tc_layer(aggs, degs, (W0_r0, W0_r1, W0_r2), (b0_r0, b0_r1, b0_r2))
    [agg] = _sc_l1(h, src, dst, z, o)
    aggs = (agg[0], agg[1], agg[2])
    return _tc_layer(aggs, degs, (W1_r0, W1_r1, W1_r2), (b1_r0, b1_r1, b1_r2))


# speedup vs baseline: 1.0467x; 1.0467x over previous
import functools

import jax
import jax.numpy as jnp
from jax import lax
from jax.experimental import pallas as pl
from jax.experimental.pallas import tpu as pltpu
from jax.experimental.pallas import tpu_sc as plsc

N = 10000
D = 256
HW = 128
E = 64000

NC = 2
NS = 16
HALF = N // NC
ACC_ROWS = 5120
GARBAGE = HALF
ZROWS = ACC_ROWS // NS
EPT = E // NS
K = 80
NCHUNK = EPT // K
OUT_T = 5
OUT_ROWS = HALF // OUT_T
NBUF = 2


def _sc_segsum(compute_deg):
    mesh = plsc.VectorSubcoreMesh(core_axis_name="c", subcore_axis_name="s")
    out_type = [jax.ShapeDtypeStruct((3, N, D), jnp.float32)]
    if compute_deg:
        out_type += [jax.ShapeDtypeStruct((3, N, HW), jnp.float32)]
    scratch = [
        pltpu.VMEM((NCHUNK, K), jnp.int32),
        pltpu.VMEM((NCHUNK, K), jnp.int32),
        pltpu.VMEM((NCHUNK, K), jnp.int32),
    ] + [pltpu.VMEM((K, HW), jnp.float32) for _ in range(NBUF)] + [
        pltpu.VMEM((K, HW), jnp.float32),
        pltpu.VMEM_SHARED((ACC_ROWS, HW), jnp.float32),
    ] + [pltpu.SemaphoreType.DMA for _ in range(2 * NBUF)]

    @functools.partial(pl.kernel, mesh=mesh, out_type=out_type,
                       scratch_types=scratch)
    def k(feat_h, src_h, dst_h, z_h, o_h, *rest):
        agg_h = rest[0]
        deg_h = rest[1] if compute_deg else None
        sc = rest[2 if compute_deg else 1:]
        srcv, dstv, ldv = sc[0:3]
        rbufs = sc[3:3 + NBUF]
        onesb, acc = sc[3 + NBUF:5 + NBUF]
        gsems = sc[5 + NBUF:5 + 2 * NBUF]
        ssems = sc[5 + 2 * NBUF:5 + 3 * NBUF]

        c = lax.axis_index("c")
        s = lax.axis_index("s")
        lo = c * HALF
        bufs = tuple(zip(rbufs, gsems, ssems))
        dummy = feat_h.at[pl.ds(0, K), pl.ds(0, HW)]

        def zero_acc():
            pltpu.sync_copy(z_h, acc.at[pl.ds(s * ZROWS, ZROWS)])

        WIN = 8

        def deg_pass(r, nt):
            zero_acc()
            plsc.subcore_barrier()

            def dbody(i, carry):
                pltpu.async_copy(onesb, acc.at[ldv.at[i]], ssems[0], add=True)

                @pl.when(i >= WIN)
                def _():
                    pltpu.make_async_copy(dummy, onesb, ssems[0]).wait()

                return carry

            lax.fori_loop(0, nt, dbody, 0)

            def ddrain(i, carry):
                pltpu.make_async_copy(dummy, onesb, ssems[0]).wait()
                return carry

            lax.fori_loop(0, WIN, ddrain, 0)
            plsc.subcore_barrier()

            @pl.when(s < OUT_T)
            def _():
                r0 = pl.multiple_of(s * OUT_ROWS, 8)
                pltpu.sync_copy(acc.at[pl.ds(r0, OUT_ROWS)],
                                deg_h.at[r, pl.ds(lo + r0, OUT_ROWS)])

            plsc.subcore_barrier()

        if compute_deg:
            pltpu.sync_copy(o_h, onesb)

        for r in range(3):
            pltpu.sync_copy(src_h.at[r, s], srcv)
            pltpu.sync_copy(dst_h.at[r, s], dstv)

            def routemk(i, carry2):
                for g in range(K // 16):
                    dv = dstv[i, pl.ds(g * 16, 16)]
                    m = (dv >= lo) & (dv < lo + HALF)
                    ldv[i, pl.ds(g * 16, 16)] = jnp.where(m, dv - lo, GARBAGE)
                return carry2

            lax.fori_loop(0, NCHUNK, routemk, 0)
            nt = NCHUNK

            if compute_deg:
                deg_pass(r, nt)

            for cb in (0, HW):
                def issue_gather(i, buf):
                    r_, gsem, _ = buf
                    pltpu.async_copy(feat_h.at[srcv.at[i], pl.ds(cb, HW)],
                                     r_, gsem)

                def wait_gather(buf):
                    r_, gsem, _ = buf
                    pltpu.make_async_copy(dummy, r_, gsem).wait()

                def issue_scatter(i, buf):
                    r_, _, ssem = buf
                    pltpu.async_copy(r_, acc.at[ldv.at[i]], ssem, add=True)

                def wait_scatter(buf):
                    r_, _, ssem = buf
                    pltpu.make_async_copy(dummy, r_, ssem).wait()

                zero_acc()
                plsc.subcore_barrier()

                for p in range(NBUF):
                    issue_gather(p, bufs[p])

                def body(j, carry):
                    for p in range(NBUF):
                        i = NBUF * j + p
                        wait_gather(bufs[p])
                        issue_scatter(i, bufs[p])

                        @pl.when(i + NBUF < nt)
                        def _():
                            wait_scatter(bufs[p])
                            issue_gather(i + NBUF, bufs[p])

                    return carry

                lax.fori_loop(0, lax.div(nt, NBUF), body, 0)
                for p in range(NBUF):
                    wait_scatter(bufs[p])
                plsc.subcore_barrier()

                @pl.when(s < OUT_T)
                def _():
                    r0 = pl.multiple_of(s * OUT_ROWS, 8)
                    pltpu.sync_copy(acc.at[pl.ds(r0, OUT_ROWS)],
                                    agg_h.at[r, pl.ds(lo + r0, OUT_ROWS),
                                             pl.ds(cb, HW)])

                plsc.subcore_barrier()

    return k


_sc_l0 = _sc_segsum(compute_deg=True)
_sc_l1 = _sc_segsum(compute_deg=False)


BLK = 512
_GRID = (N + BLK - 1) // BLK


def _tc_body(a, g, w0, w1, w2, b0, b1, b2, o):
    acc = b0[...] + b1[...] + b2[...]
    for r, w in enumerate((w0, w1, w2)):
        inv = 1.0 / jnp.maximum(g[r][:, 0:1], 1.0)
        acc = acc + jnp.dot(a[r] * inv, w[...],
                            preferred_element_type=jnp.float32)
    o[...] = jnp.maximum(acc, 0.0)


def _tc_layer(agg3, deg3, ws, bs):
    row3 = lambda i: (0, i, 0)
    fix = lambda i: (0, 0)
    return pl.pallas_call(
        _tc_body,
        grid=(_GRID,),
        in_specs=[
            pl.BlockSpec((3, BLK, D), row3),
            pl.BlockSpec((3, BLK, HW), row3),
            pl.BlockSpec((D, D), fix), pl.BlockSpec((D, D), fix),
            pl.BlockSpec((D, D), fix),
            pl.BlockSpec((1, D), fix), pl.BlockSpec((1, D), fix),
            pl.BlockSpec((1, D), fix),
        ],
        out_specs=pl.BlockSpec((BLK, D), lambda i: (i, 0)),
        out_shape=jax.ShapeDtypeStruct((N, D), jnp.float32),
    )(agg3, deg3, *ws, bs[0].reshape(1, D), bs[1].reshape(1, D),
      bs[2].reshape(1, D))


def kernel(x, edge_index_r0, edge_index_r1, edge_index_r2,
           W0_r0, b0_r0, W0_r1, b0_r1, W0_r2, b0_r2,
           W1_r0, b1_r0, W1_r1, b1_r1, W1_r2, b1_r2):
    ei = jnp.stack([edge_index_r0, edge_index_r1, edge_index_r2])
    src = ei[:, 0].reshape(3, NS, NCHUNK, K)
    dst = ei[:, 1].reshape(3, NS, NCHUNK, K)
    z = jnp.zeros((ZROWS, HW), jnp.float32)
    o = jnp.ones((K, HW), jnp.float32)

    agg, deg = _sc_l0(x, src, dst, z, o)
    h = _tc_layer(agg, deg, (W0_r0, W0_r1, W0_r2), (b0_r0, b0_r1, b0_r2))
    [agg] = _sc_l1(h, src, dst, z, o)
    return _tc_layer(agg, deg, (W1_r0, W1_r1, W1_r2), (b1_r0, b1_r1, b1_r2))
